# baseline (speedup 1.0000x reference)
import functools

import jax
import jax.numpy as jnp
from jax import lax
from jax.experimental import pallas as pl
from jax.experimental.pallas import tpu as pltpu
from jax.experimental.pallas import tpu_sc as plsc

_NW = 5
_MS = 125
_MU = 5000
_MUP = 5120
_CAP = 640
_W2 = 128 + _CAP
_B = 4
_Q = 75
_QT = 15
_C = 64
_EPS = 1e-12


def _rownorm(x):
    n = jnp.sqrt(jnp.sum(x * x, axis=1, keepdims=True))
    return x / jnp.maximum(n, _EPS)


def _s1_body(u_ref, s_ref, idx_ref, cnt_ref):
    bp = pl.program_id(0)
    u = u_ref[0]
    s = s_ref[0]
    un = _rownorm(u)
    sn = _rownorm(s)
    S = lax.dot_general(un, sn, (((1,), (1,)), ((), ())),
                        preferred_element_type=jnp.float32)
    row1 = lax.broadcasted_iota(jnp.int32, (_MUP, 1), 0)
    col1 = lax.broadcasted_iota(jnp.int32, (1, _NW * _MS + 15), 1)
    neg = jnp.float32(-jnp.inf)
    S = jnp.where(col1 < _NW * _MS, S, neg)
    S = jnp.where(row1 < _MU, S, neg)
    u_near = jnp.argmax(S, axis=1, keepdims=True).astype(jnp.int32)
    s_near = jnp.argmax(S, axis=0, keepdims=True).astype(jnp.int32)
    best = jnp.max(S[:, 0:_MS], axis=1, keepdims=True)
    bi = jnp.zeros((_MUP, 1), jnp.int32)
    for n in range(1, _NW):
        cm = jnp.max(S[:, n * _MS:(n + 1) * _MS], axis=1, keepdims=True)
        upd = cm > best
        bi = jnp.where(upd, n, bi)
        best = jnp.where(upd, cm, best)
    oh = u_near == col1
    mutual = jnp.sum(jnp.where(oh, s_near, 0), axis=1, keepdims=True)
    selected = (mutual == row1) & (row1 < _MU)
    lane128 = lax.broadcasted_iota(jnp.int32, (1, 128), 1)
    hio8 = lax.broadcasted_iota(jnp.int32, (1, 8), 1)
    gidxf = (row1 + bp * _MUP).astype(jnp.float32)
    idxs = []
    cnts = []
    for n in range(_NW):
        mn_b = selected & (bi == n)
        mn = jnp.where(mn_b, 1, 0)
        inc = mn
        k = 1
        while k < _MUP:
            inc = inc + jnp.concatenate(
                [jnp.zeros((k, 1), jnp.int32), inc[:_MUP - k]], axis=0)
            k *= 2
        pos_n = jnp.where(mn_b, inc - 1, -1)
        hi = pos_n >> 7
        lo = pos_n & 127
        V = jnp.where(hi == hio8, gidxf, 0.0)
        OHlo = jnp.where(lo == lane128, 1.0, 0.0)
        idx_n = lax.dot_general(V, OHlo, (((0,), (0,)), ((), ())),
                                precision=lax.Precision.HIGHEST,
                                preferred_element_type=jnp.float32)
        idxs.append(idx_n[0:_CAP // 128].astype(jnp.int32)[None])
        cnts.append(jnp.full((1, 16), jnp.sum(mn), jnp.int32))
    idx_ref[...] = jnp.concatenate(idxs, axis=0)[None]
    cnt_ref[...] = jnp.concatenate(cnts, axis=0)[None]


def _s1_call(u_pad, s_pad):
    return pl.pallas_call(
        _s1_body,
        grid=(_B,),
        in_specs=[
            pl.BlockSpec((1, _MUP, _C), lambda b: (b, 0, 0)),
            pl.BlockSpec((1, _NW * _MS + 15, _C), lambda b: (b, 0, 0)),
        ],
        out_specs=[
            pl.BlockSpec((1, _NW, _CAP // 128, 128), lambda b: (b, 0, 0, 0)),
            pl.BlockSpec((1, _NW, 16), lambda b: (b, 0, 0)),
        ],
        out_shape=[
            jax.ShapeDtypeStruct((_B, _NW, _CAP // 128, 128), jnp.int32),
            jax.ShapeDtypeStruct((_B, _NW, 16), jnp.int32),
        ],
    )(u_pad, s_pad)


def _sc_body(idx_hbm, cnt_hbm, uflat_hbm, dual_hbm, cnt_v, idx2_v, rows_v,
             sem):
    wid = lax.axis_index("s") * 2 + lax.axis_index("c")

    @pl.when(wid < _B * _NW)
    def _():
        pltpu.sync_copy(idx_hbm.at[wid], idx2_v)
        pltpu.sync_copy(cnt_hbm.at[wid], cnt_v)
        cnt = jnp.max(cnt_v[...])
        for j in range(_CAP // 128):
            @pl.when(j * 128 < cnt)
            def _(j=j):
                pltpu.async_copy(uflat_hbm.at[idx2_v.at[j]],
                                 rows_v.at[pl.ds(j * 128, 128)],
                                 sem).wait()
                pltpu.sync_copy(rows_v.at[pl.ds(j * 128, 128)],
                                dual_hbm.at[wid].at[pl.ds(j * 128, 128)])


@functools.lru_cache(maxsize=1)
def _sc_compact_fn():
    mesh = plsc.VectorSubcoreMesh(core_axis_name="c", subcore_axis_name="s")
    return pl.kernel(
        _sc_body,
        mesh=mesh,
        out_type=jax.ShapeDtypeStruct((_B * _NW, _CAP, 128), jnp.float32),
        scratch_types=[
            pltpu.VMEM((16,), jnp.int32),
            pltpu.VMEM((_CAP // 128, 128), jnp.int32),
            pltpu.VMEM((_CAP, 128), jnp.float32),
            pltpu.SemaphoreType.DMA,
        ],
        compiler_params=pltpu.CompilerParams(needs_layout_passes=False),
    )


def _s2_body(counts_ref, qy_ref, qf_ref, sup_ref, dual_ref, out_ref):
    bi = pl.program_id(0)
    qt = pl.program_id(1)
    L = counts_ref[0, 0]
    for b_ in range(_B):
        for n_ in range(_NW):
            L = jnp.maximum(L, counts_ref[b_, n_])
    q = qf_ref[0].reshape(_QT * 32, _C)
    qn = _rownorm(q)
    pieces = []
    for n in range(_NW):
        pieces.append(sup_ref[0, n])
        pieces.append(dual_ref[0, n, :, 0:_C])
    scat = _rownorm(jnp.concatenate(pieces, axis=0))
    S = lax.dot_general(qn, scat, (((1,), (1,)), ((), ())),
                        preferred_element_type=jnp.float32)
    colg = lax.broadcasted_iota(jnp.int32, (1, _NW * _W2), 1)
    jloc = colg % _W2
    valid = (jloc < _MS) | ((jloc >= 128) & (jloc < 128 + L))
    cntcol = jnp.concatenate(
        [jnp.full((1, _W2), counts_ref[bi, n], jnp.int32)
         for n in range(_NW)], axis=1)
    zerocols = (jloc >= 128) & (jloc < 128 + L) & ((jloc - 128) >= cntcol)
    rowp = lax.broadcasted_iota(jnp.int32, (_QT * 32, 1), 0)
    neg = jnp.float32(-jnp.inf)
    S = jnp.where(valid, S, neg)
    S = jnp.where(zerocols, jnp.float32(0.0), S)
    S = jnp.where((rowp % 32) >= 25, neg, S)
    rms = [jnp.max(S[:, n * _W2:(n + 1) * _W2], axis=1, keepdims=True)
           for n in range(_NW)]
    row32 = lax.broadcasted_iota(jnp.int32, (32, 1), 0)
    nll_sum = jnp.float32(0.0)
    for k in range(_QT):
        Sk = S[k * 32:(k + 1) * 32, :]
        cmax = jnp.max(Sk, axis=0, keepdims=True)
        carg = jnp.min(jnp.where(Sk == cmax, row32, 99), axis=0,
                       keepdims=True)
        qnear = jnp.argmax(Sk, axis=1, keepdims=True).astype(jnp.int32)
        mutual = jnp.sum(jnp.where(qnear == colg, carg, 0), axis=1,
                         keepdims=True)
        qmask = (mutual == row32) & (row32 < 25)
        logits = []
        for n in range(_NW):
            rmk = rms[n][k * 32:(k + 1) * 32, :]
            qv = jnp.sum(jnp.where(qmask, rmk, 0.0))
            logits.append(qv / 2.0)
        m = logits[0]
        for n in range(1, _NW):
            m = jnp.maximum(m, logits[n])
        sexp = jnp.float32(0.0)
        for n in range(_NW):
            sexp = sexp + jnp.exp(logits[n] - m)
        lse = m + jnp.log(sexp)
        y = qy_ref[bi, qt * _QT + k]
        pick = jnp.float32(0.0)
        for n in range(_NW):
            pick = pick + jnp.where(y == n, logits[n], 0.0)
        nll_sum = nll_sum + (lse - pick)

    @pl.when((bi == 0) & (qt == 0))
    def _():
        out_ref[...] = jnp.zeros((1, 1), jnp.float32)

    out_ref[...] = out_ref[...] + (nll_sum / (_B * _Q)).reshape(1, 1)


def _s2_call(counts, qy, qf_pad, sup_pad, dual):
    return pl.pallas_call(
        _s2_body,
        grid=(_B, _Q // _QT),
        in_specs=[
            pl.BlockSpec(memory_space=pltpu.SMEM),
            pl.BlockSpec(memory_space=pltpu.SMEM),
            pl.BlockSpec((1, _QT, 32, _C), lambda b, t: (b, t, 0, 0)),
            pl.BlockSpec((1, _NW, 128, _C), lambda b, t: (b, 0, 0, 0)),
            pl.BlockSpec((1, _NW, _CAP, 128), lambda b, t: (b, 0, 0, 0)),
        ],
        out_specs=pl.BlockSpec((1, 1), lambda b, t: (0, 0)),
        out_shape=jax.ShapeDtypeStruct((1, 1), jnp.float32),
    )(counts, qy, qf_pad, sup_pad, dual)


def kernel(support_xf, support_y, query_xf, query_y, unlabeled_xf):
    u_feats = unlabeled_xf.reshape(_B, 200, _C, 25).transpose(0, 1, 3, 2)
    u_feats = u_feats.reshape(_B, _MU, _C)
    u_pad = jnp.pad(u_feats, ((0, 0), (0, _MUP - _MU), (0, 0)))
    sup4 = support_xf.reshape(_B, _NW, _NW, _C, 25).transpose(0, 1, 3, 2, 4)
    sup4 = sup4.reshape(_B, _NW, _C, _MS)
    s_cols = sup4.transpose(0, 1, 3, 2).reshape(_B, _NW * _MS, _C)
    s_pad = jnp.pad(s_cols, ((0, 0), (0, 15), (0, 0)))

    idx4, counts16 = _s1_call(u_pad, s_pad)
    idx20 = idx4.reshape(_B * _NW, _CAP // 128, 128)
    cnt20 = counts16.reshape(_B * _NW, 16)
    counts = counts16[:, :, 0]
    uflat = jnp.pad(u_pad, ((0, 0), (0, 0), (0, 128 - _C)))
    uflat = uflat.reshape(_B * _MUP, 128)
    dual20 = _sc_compact_fn()(idx20, cnt20, uflat)
    dual = dual20.reshape(_B, _NW, _CAP, 128)

    sup_rows = sup4.transpose(0, 1, 3, 2)
    sup_pad = jnp.pad(sup_rows, ((0, 0), (0, 0), (0, 3), (0, 0)))
    qf = query_xf.reshape(_B, _Q, _C, 25).transpose(0, 1, 3, 2)
    qf_pad = jnp.pad(qf, ((0, 0), (0, 0), (0, 7), (0, 0)))

    out = _s2_call(counts, query_y, qf_pad, sup_pad, dual)
    return out.reshape(())


# speedup vs baseline: 1.1369x; 1.1369x over previous
import functools

import jax
import jax.numpy as jnp
from jax import lax
from jax.experimental import pallas as pl
from jax.experimental.pallas import tpu as pltpu
from jax.experimental.pallas import tpu_sc as plsc

_NW = 5
_MS = 125
_MU = 5000
_MUP = 5120
_CAP = 128
_W2 = 128 + _CAP
_B = 4
_Q = 75
_QT = 15
_C = 64
_EPS = 1e-12


def _rownorm(x):
    n = jnp.sqrt(jnp.sum(x * x, axis=1, keepdims=True))
    return x / jnp.maximum(n, _EPS)


def _s1_body(u_ref, s_ref, idx_ref, cnt_ref):
    bp = pl.program_id(0)
    u = u_ref[0]
    s = s_ref[0]
    un = _rownorm(u)
    sn = _rownorm(s)
    S = lax.dot_general(un, sn, (((1,), (1,)), ((), ())),
                        preferred_element_type=jnp.float32)
    row1 = lax.broadcasted_iota(jnp.int32, (_MUP, 1), 0)
    col1 = lax.broadcasted_iota(jnp.int32, (1, _NW * _MS + 15), 1)
    neg = jnp.float32(-jnp.inf)
    S = jnp.where(col1 < _NW * _MS, S, neg)
    S = jnp.where(row1 < _MU, S, neg)
    u_near = jnp.argmax(S, axis=1, keepdims=True).astype(jnp.int32)
    s_near = jnp.argmax(S, axis=0, keepdims=True).astype(jnp.int32)
    best = jnp.max(S[:, 0:_MS], axis=1, keepdims=True)
    bi = jnp.zeros((_MUP, 1), jnp.int32)
    for n in range(1, _NW):
        cm = jnp.max(S[:, n * _MS:(n + 1) * _MS], axis=1, keepdims=True)
        upd = cm > best
        bi = jnp.where(upd, n, bi)
        best = jnp.where(upd, cm, best)
    oh = u_near == col1
    mutual = jnp.sum(jnp.where(oh, s_near, 0), axis=1, keepdims=True)
    selected = (mutual == row1) & (row1 < _MU)
    lane128 = lax.broadcasted_iota(jnp.int32, (1, 128), 1)
    hio8 = lax.broadcasted_iota(jnp.int32, (1, 8), 1)
    gidxf = (row1 + bp * _MUP).astype(jnp.float32)
    idxs = []
    cnts = []
    for n in range(_NW):
        mn_b = selected & (bi == n)
        mn = jnp.where(mn_b, 1, 0)
        inc = mn
        k = 1
        while k < _MUP:
            inc = inc + jnp.concatenate(
                [jnp.zeros((k, 1), jnp.int32), inc[:_MUP - k]], axis=0)
            k *= 2
        pos_n = jnp.where(mn_b, inc - 1, -1)
        hi = pos_n >> 7
        lo = pos_n & 127
        V = jnp.where(hi == hio8, gidxf, 0.0)
        OHlo = jnp.where(lo == lane128, 1.0, 0.0)
        idx_n = lax.dot_general(V, OHlo, (((0,), (0,)), ((), ())),
                                precision=lax.Precision.HIGHEST,
                                preferred_element_type=jnp.float32)
        idxs.append(idx_n[0:_CAP // 128].astype(jnp.int32)[None])
        cnts.append(jnp.full((1, 16), jnp.sum(mn), jnp.int32))
    idx_ref[...] = jnp.concatenate(idxs, axis=0)[None]
    cnt_ref[...] = jnp.concatenate(cnts, axis=0)[None]


def _s1_call(u_pad, s_pad):
    return pl.pallas_call(
        _s1_body,
        grid=(_B,),
        in_specs=[
            pl.BlockSpec((1, _MUP, _C), lambda b: (b, 0, 0)),
            pl.BlockSpec((1, _NW * _MS + 15, _C), lambda b: (b, 0, 0)),
        ],
        out_specs=[
            pl.BlockSpec((1, _NW, _CAP // 128, 128), lambda b: (b, 0, 0, 0)),
            pl.BlockSpec((1, _NW, 16), lambda b: (b, 0, 0)),
        ],
        out_shape=[
            jax.ShapeDtypeStruct((_B, _NW, _CAP // 128, 128), jnp.int32),
            jax.ShapeDtypeStruct((_B, _NW, 16), jnp.int32),
        ],
    )(u_pad, s_pad)


def _sc_body(idx_hbm, cnt_hbm, uflat_hbm, dual_hbm, cnt_v, idx2_v, rows_v,
             sem):
    wid = lax.axis_index("s") * 2 + lax.axis_index("c")

    @pl.when(wid < _B * _NW)
    def _():
        pltpu.sync_copy(idx_hbm.at[wid], idx2_v)
        pltpu.sync_copy(cnt_hbm.at[wid], cnt_v)
        cnt = jnp.max(cnt_v[...])
        for j in range(_CAP // 128):
            @pl.when(j * 128 < cnt)
            def _(j=j):
                pltpu.async_copy(uflat_hbm.at[idx2_v.at[j]],
                                 rows_v.at[pl.ds(j * 128, 128)],
                                 sem).wait()
                pltpu.sync_copy(rows_v.at[pl.ds(j * 128, 128)],
                                dual_hbm.at[wid].at[pl.ds(j * 128, 128)])


@functools.lru_cache(maxsize=1)
def _sc_compact_fn():
    mesh = plsc.VectorSubcoreMesh(core_axis_name="c", subcore_axis_name="s")
    return pl.kernel(
        _sc_body,
        mesh=mesh,
        out_type=jax.ShapeDtypeStruct((_B * _NW, _CAP, 128), jnp.float32),
        scratch_types=[
            pltpu.VMEM((16,), jnp.int32),
            pltpu.VMEM((_CAP // 128, 128), jnp.int32),
            pltpu.VMEM((_CAP, 128), jnp.float32),
            pltpu.SemaphoreType.DMA,
        ],
        compiler_params=pltpu.CompilerParams(needs_layout_passes=False),
    )


def _s2_body(counts_ref, qy_ref, qf_ref, sup_ref, dual_ref, out_ref):
    bi = pl.program_id(0)
    qt = pl.program_id(1)
    L = counts_ref[0, 0]
    for b_ in range(_B):
        for n_ in range(_NW):
            L = jnp.maximum(L, counts_ref[b_, n_])
    q = qf_ref[0].reshape(_QT * 32, _C)
    qn = _rownorm(q)
    pieces = []
    for n in range(_NW):
        pieces.append(sup_ref[0, n])
        pieces.append(dual_ref[0, n, :, 0:_C])
    scat = _rownorm(jnp.concatenate(pieces, axis=0))
    S = lax.dot_general(qn, scat, (((1,), (1,)), ((), ())),
                        preferred_element_type=jnp.float32)
    colg = lax.broadcasted_iota(jnp.int32, (1, _NW * _W2), 1)
    jloc = colg % _W2
    valid = (jloc < _MS) | ((jloc >= 128) & (jloc < 128 + L))
    cntcol = jnp.concatenate(
        [jnp.full((1, _W2), counts_ref[bi, n], jnp.int32)
         for n in range(_NW)], axis=1)
    zerocols = (jloc >= 128) & (jloc < 128 + L) & ((jloc - 128) >= cntcol)
    rowp = lax.broadcasted_iota(jnp.int32, (_QT * 32, 1), 0)
    neg = jnp.float32(-jnp.inf)
    S = jnp.where(valid, S, neg)
    S = jnp.where(zerocols, jnp.float32(0.0), S)
    S = jnp.where((rowp % 32) >= 25, neg, S)
    rms = [jnp.max(S[:, n * _W2:(n + 1) * _W2], axis=1, keepdims=True)
           for n in range(_NW)]
    row32 = lax.broadcasted_iota(jnp.int32, (32, 1), 0)
    nll_sum = jnp.float32(0.0)
    for k in range(_QT):
        Sk = S[k * 32:(k + 1) * 32, :]
        cmax = jnp.max(Sk, axis=0, keepdims=True)
        carg = jnp.min(jnp.where(Sk == cmax, row32, 99), axis=0,
                       keepdims=True)
        qnear = jnp.argmax(Sk, axis=1, keepdims=True).astype(jnp.int32)
        mutual = jnp.sum(jnp.where(qnear == colg, carg, 0), axis=1,
                         keepdims=True)
        qmask = (mutual == row32) & (row32 < 25)
        logits = []
        for n in range(_NW):
            rmk = rms[n][k * 32:(k + 1) * 32, :]
            qv = jnp.sum(jnp.where(qmask, rmk, 0.0))
            logits.append(qv / 2.0)
        m = logits[0]
        for n in range(1, _NW):
            m = jnp.maximum(m, logits[n])
        sexp = jnp.float32(0.0)
        for n in range(_NW):
            sexp = sexp + jnp.exp(logits[n] - m)
        lse = m + jnp.log(sexp)
        y = qy_ref[bi, qt * _QT + k]
        pick = jnp.float32(0.0)
        for n in range(_NW):
            pick = pick + jnp.where(y == n, logits[n], 0.0)
        nll_sum = nll_sum + (lse - pick)

    @pl.when((bi == 0) & (qt == 0))
    def _():
        out_ref[...] = jnp.zeros((1, 1), jnp.float32)

    out_ref[...] = out_ref[...] + (nll_sum / (_B * _Q)).reshape(1, 1)


def _s2_call(counts, qy, qf_pad, sup_pad, dual):
    return pl.pallas_call(
        _s2_body,
        grid=(_B, _Q // _QT),
        in_specs=[
            pl.BlockSpec(memory_space=pltpu.SMEM),
            pl.BlockSpec(memory_space=pltpu.SMEM),
            pl.BlockSpec((1, _QT, 32, _C), lambda b, t: (b, t, 0, 0)),
            pl.BlockSpec((1, _NW, 128, _C), lambda b, t: (b, 0, 0, 0)),
            pl.BlockSpec((1, _NW, _CAP, 128), lambda b, t: (b, 0, 0, 0)),
        ],
        out_specs=pl.BlockSpec((1, 1), lambda b, t: (0, 0)),
        out_shape=jax.ShapeDtypeStruct((1, 1), jnp.float32),
    )(counts, qy, qf_pad, sup_pad, dual)


def kernel(support_xf, support_y, query_xf, query_y, unlabeled_xf):
    u_feats = unlabeled_xf.reshape(_B, 200, _C, 25).transpose(0, 1, 3, 2)
    u_feats = u_feats.reshape(_B, _MU, _C)
    u_pad = jnp.pad(u_feats, ((0, 0), (0, _MUP - _MU), (0, 0)))
    sup4 = support_xf.reshape(_B, _NW, _NW, _C, 25).transpose(0, 1, 3, 2, 4)
    sup4 = sup4.reshape(_B, _NW, _C, _MS)
    s_cols = sup4.transpose(0, 1, 3, 2).reshape(_B, _NW * _MS, _C)
    s_pad = jnp.pad(s_cols, ((0, 0), (0, 15), (0, 0)))

    idx4, counts16 = _s1_call(u_pad, s_pad)
    idx20 = idx4.reshape(_B * _NW, _CAP // 128, 128)
    cnt20 = counts16.reshape(_B * _NW, 16)
    counts = counts16[:, :, 0]
    uflat = jnp.pad(u_pad, ((0, 0), (0, 0), (0, 128 - _C)))
    uflat = uflat.reshape(_B * _MUP, 128)
    dual20 = _sc_compact_fn()(idx20, cnt20, uflat)
    dual = dual20.reshape(_B, _NW, _CAP, 128)

    sup_rows = sup4.transpose(0, 1, 3, 2)
    sup_pad = jnp.pad(sup_rows, ((0, 0), (0, 0), (0, 3), (0, 0)))
    qf = query_xf.reshape(_B, _Q, _C, 25).transpose(0, 1, 3, 2)
    qf_pad = jnp.pad(qf, ((0, 0), (0, 0), (0, 7), (0, 0)))

    out = _s2_call(counts, query_y, qf_pad, sup_pad, dual)
    return out.reshape(())
